# speedup vs baseline: 1.4758x; 1.4758x over previous
import jax
import jax.numpy as jnp
from jax.experimental import pallas as pl
from jax.experimental.pallas import tpu as pltpu

S_BLK = 256


def _add_body(x_ref, p_ref, o_ref):
    o_ref[...] = x_ref[...] + p_ref[...]


def kernel(inputs, pos_table):
    B, S, D = inputs.shape
    grid = (S // S_BLK, B)
    return pl.pallas_call(
        _add_body,
        grid=grid,
        in_specs=[
            pl.BlockSpec((1, S_BLK, D), lambda s, b: (b, s, 0)),
            pl.BlockSpec((S_BLK, D), lambda s, b: (s, 0)),
        ],
        out_specs=pl.BlockSpec((1, S_BLK, D), lambda s, b: (b, s, 0)),
        out_shape=jax.ShapeDtypeStruct((B, S, D), inputs.dtype),
        compiler_params=pltpu.CompilerParams(
            dimension_semantics=("parallel", "parallel"),
        ),
    )(inputs, pos_table)


# speedup vs baseline: 1.9467x; 1.3191x over previous
import jax
import jax.numpy as jnp
from jax.experimental import pallas as pl
from jax.experimental.pallas import tpu as pltpu

S_BLK = 512


def _add_body(x_ref, p_ref, o_ref):
    o_ref[...] = x_ref[...] + p_ref[...]


def kernel(inputs, pos_table):
    B, S, D = inputs.shape
    grid = (S // S_BLK, B)
    return pl.pallas_call(
        _add_body,
        grid=grid,
        in_specs=[
            pl.BlockSpec((1, S_BLK, D), lambda s, b: (b, s, 0)),
            pl.BlockSpec((S_BLK, D), lambda s, b: (s, 0)),
        ],
        out_specs=pl.BlockSpec((1, S_BLK, D), lambda s, b: (b, s, 0)),
        out_shape=jax.ShapeDtypeStruct((B, S, D), inputs.dtype),
        compiler_params=pltpu.CompilerParams(
            dimension_semantics=("parallel", "parallel"),
        ),
    )(inputs, pos_table)


# speedup vs baseline: 2.1026x; 1.0800x over previous
import jax
import jax.numpy as jnp
from jax.experimental import pallas as pl
from jax.experimental.pallas import tpu as pltpu

S_BLK = 1024


def _add_body(x_ref, p_ref, o_ref):
    o_ref[...] = x_ref[...] + p_ref[...]


def kernel(inputs, pos_table):
    B, S, D = inputs.shape
    grid = (S // S_BLK, B)
    return pl.pallas_call(
        _add_body,
        grid=grid,
        in_specs=[
            pl.BlockSpec((1, S_BLK, D), lambda s, b: (b, s, 0)),
            pl.BlockSpec((S_BLK, D), lambda s, b: (s, 0)),
        ],
        out_specs=pl.BlockSpec((1, S_BLK, D), lambda s, b: (b, s, 0)),
        out_shape=jax.ShapeDtypeStruct((B, S, D), inputs.dtype),
        compiler_params=pltpu.CompilerParams(
            dimension_semantics=("parallel", "parallel"),
        ),
    )(inputs, pos_table)


# speedup vs baseline: 2.2829x; 1.0858x over previous
import jax
import jax.numpy as jnp
from jax.experimental import pallas as pl
from jax.experimental.pallas import tpu as pltpu

S_BLK = 2048


def _add_body(x_ref, p_ref, o_ref):
    o_ref[...] = x_ref[...] + p_ref[...]


def kernel(inputs, pos_table):
    B, S, D = inputs.shape
    grid = (S // S_BLK, B)
    return pl.pallas_call(
        _add_body,
        grid=grid,
        in_specs=[
            pl.BlockSpec((1, S_BLK, D), lambda s, b: (b, s, 0)),
            pl.BlockSpec((S_BLK, D), lambda s, b: (s, 0)),
        ],
        out_specs=pl.BlockSpec((1, S_BLK, D), lambda s, b: (b, s, 0)),
        out_shape=jax.ShapeDtypeStruct((B, S, D), inputs.dtype),
        compiler_params=pltpu.CompilerParams(
            dimension_semantics=("parallel", "parallel"),
        ),
    )(inputs, pos_table)
